# baseline (speedup 1.0000x reference)
import functools

import jax
import jax.numpy as jnp
from jax import lax
from jax.experimental import pallas as pl
from jax.experimental.pallas import tpu as pltpu
from jax.experimental.pallas import tpu_sc as plsc

BATCH = 16384
DIM = 64
L = 16
NTAB = 1000
PACK = 2 * DIM

_info = plsc.get_sparse_core_info()
NC, NS = _info.num_cores, _info.num_subcores
NW = NC * NS
B_PER_W = BATCH // NW
CHUNK = 64
N_CHUNKS = B_PER_W // CHUNK


def _body(s_id, r_id, o_id, t_id, ent2, rel_flat, tim_flat, out,
          sidx, ridx, oidx, tidx, spk, opk, tab, rt, srow, orow, outv,
          sem, semt):
    wid = lax.axis_index("s") * NC + lax.axis_index("c")
    base = wid * B_PER_W
    lanes = lax.iota(jnp.int32, L)

    cp_tab = pltpu.async_copy(rel_flat, tab, semt)
    pltpu.sync_copy(s_id.at[pl.ds(base, B_PER_W)], sidx)
    pltpu.sync_copy(r_id.at[pl.ds(base, B_PER_W)], ridx)
    pltpu.sync_copy(o_id.at[pl.ds(base, B_PER_W)], oidx)
    pltpu.sync_copy(t_id.at[pl.ds(base, B_PER_W)], tidx)

    def pack_group(g, _):
        sl = pl.ds(g * L, L)
        spk[sl] = sidx[sl] >> 1
        opk[sl] = oidx[sl] >> 1
        return 0

    lax.fori_loop(0, B_PER_W // L, pack_group, 0)

    cp_s0 = pltpu.async_copy(ent2.at[spk.at[pl.ds(0, CHUNK)]], srow, sem)
    cp_o0 = pltpu.async_copy(ent2.at[opk.at[pl.ds(0, CHUNK)]], orow, sem)
    cp_tab.wait()

    def rel_group(g, _):
        tv = ridx[pl.ds(g * L, L)] * DIM

        def d_body(d, _):
            rt[d, pl.ds(g * L, L)] = plsc.load_gather(tab, [tv + d])
            return 0

        lax.fori_loop(0, DIM, d_body, 0)
        return 0

    lax.fori_loop(0, B_PER_W // L, rel_group, 0)

    pltpu.sync_copy(tim_flat, tab)

    def tim_group(g, _):
        tv = tidx[pl.ds(g * L, L)] * DIM

        def d_body(d, _):
            rt[d, pl.ds(g * L, L)] += plsc.load_gather(tab, [tv + d])
            return 0

        lax.fori_loop(0, DIM, d_body, 0)
        return 0

    lax.fori_loop(0, B_PER_W // L, tim_group, 0)

    cp_s0.wait()
    cp_o0.wait()

    for c in range(N_CHUNKS):
        cb = c * CHUNK

        def score_group(g, _):
            rowv = lanes + g * L
            soff = (sidx[pl.ds(cb + g * L, L)] & 1) << 6
            ooff = (oidx[pl.ds(cb + g * L, L)] & 1) << 6

            def d_body(d, acc):
                sv = plsc.load_gather(srow, [rowv, soff + d])
                ov = plsc.load_gather(orow, [rowv, ooff + d])
                rtv = rt[d, pl.ds(cb + g * L, L)]
                diff = sv + rtv - ov
                return acc + diff * diff

            acc = lax.fori_loop(0, DIM, d_body, jnp.zeros((L,), jnp.float32))
            seed = jnp.int32(0x5F3759DF) - (plsc.bitcast(acc, jnp.int32) >> 1)
            y = plsc.bitcast(seed, jnp.float32)
            half = acc * jnp.float32(0.5)
            for _i in range(3):
                y = y * (jnp.float32(1.5) - half * y * y)
            outv[pl.ds(cb + g * L, L)] = -(acc * y)
            return 0

        lax.fori_loop(0, CHUNK // L, score_group, 0)

        if c + 1 < N_CHUNKS:
            nb = (c + 1) * CHUNK
            cp_s = pltpu.async_copy(ent2.at[spk.at[pl.ds(nb, CHUNK)]],
                                    srow, sem)
            cp_o = pltpu.async_copy(ent2.at[opk.at[pl.ds(nb, CHUNK)]],
                                    orow, sem)
            cp_s.wait()
            cp_o.wait()

    pltpu.sync_copy(outv, out.at[pl.ds(base, B_PER_W)])


_sc_call = functools.partial(
    pl.kernel,
    mesh=plsc.VectorSubcoreMesh(core_axis_name="c", subcore_axis_name="s"),
    out_type=jax.ShapeDtypeStruct((BATCH,), jnp.float32),
    compiler_params=pltpu.CompilerParams(needs_layout_passes=False),
    scratch_types=[
        pltpu.VMEM((B_PER_W,), jnp.int32),
        pltpu.VMEM((B_PER_W,), jnp.int32),
        pltpu.VMEM((B_PER_W,), jnp.int32),
        pltpu.VMEM((B_PER_W,), jnp.int32),
        pltpu.VMEM((B_PER_W,), jnp.int32),
        pltpu.VMEM((B_PER_W,), jnp.int32),
        pltpu.VMEM((NTAB * DIM,), jnp.float32),
        pltpu.VMEM((DIM, B_PER_W), jnp.float32),
        pltpu.VMEM((CHUNK, PACK), jnp.float32),
        pltpu.VMEM((CHUNK, PACK), jnp.float32),
        pltpu.VMEM((B_PER_W,), jnp.float32),
        pltpu.SemaphoreType.DMA,
        pltpu.SemaphoreType.DMA,
    ],
)(_body)


def kernel(s_id, r_id, o_id, t_id, entities, relations, times):
    return _sc_call(s_id.astype(jnp.int32), r_id.astype(jnp.int32),
                    o_id.astype(jnp.int32), t_id.astype(jnp.int32),
                    jnp.concatenate([entities[0::2], entities[1::2]], axis=1),
                    relations.reshape(-1), times.reshape(-1))


# speedup vs baseline: 17.0354x; 17.0354x over previous
import functools

import jax
import jax.numpy as jnp
from jax import lax
from jax.experimental import pallas as pl
from jax.experimental.pallas import tpu as pltpu
from jax.experimental.pallas import tpu_sc as plsc

BATCH = 16384
DIM = 64
L = 16
NTAB = 1000

_info = plsc.get_sparse_core_info()
NC, NS = _info.num_cores, _info.num_subcores
NW = NC * NS
B_PER_W = BATCH // NW
CHUNK = 64
N_CHUNKS = B_PER_W // CHUNK


def _body(s_id, r_id, o_id, t_id, ent, rel_flat, tim_flat, out,
          sidx, ridx, oidx, tidx, tab, rt, srow, orow, outv, sem, semt):
    wid = lax.axis_index("s") * NC + lax.axis_index("c")
    base = wid * B_PER_W
    lanes = lax.iota(jnp.int32, L)

    cp_tab = pltpu.async_copy(rel_flat, tab, semt)
    pltpu.sync_copy(s_id.at[pl.ds(base, B_PER_W)], sidx)
    pltpu.sync_copy(r_id.at[pl.ds(base, B_PER_W)], ridx)
    pltpu.sync_copy(o_id.at[pl.ds(base, B_PER_W)], oidx)
    pltpu.sync_copy(t_id.at[pl.ds(base, B_PER_W)], tidx)
    cp_tab.wait()

    def rel_group(g, _):
        rowv = lanes + g * L
        tv = ridx[pl.ds(g * L, L)] * DIM
        dstv = rowv * DIM

        def d_body(d, _):
            v = plsc.load_gather(tab, [tv + d])
            plsc.store_scatter(rt, [dstv + d], v)
            return 0

        lax.fori_loop(0, DIM, d_body, 0)
        return 0

    lax.fori_loop(0, B_PER_W // L, rel_group, 0)

    pltpu.sync_copy(tim_flat, tab)

    def tim_group(g, _):
        rowv = lanes + g * L
        tv = tidx[pl.ds(g * L, L)] * DIM
        dstv = rowv * DIM

        def d_body(d, _):
            v = plsc.load_gather(tab, [tv + d])
            plsc.addupdate_scatter(rt, [dstv + d], v)
            return 0

        lax.fori_loop(0, DIM, d_body, 0)
        return 0

    lax.fori_loop(0, B_PER_W // L, tim_group, 0)

    for c in range(N_CHUNKS):
        cb = c * CHUNK

        def fetch(g, _):
            sv_idx = sidx[pl.ds(cb + g * L, L)]
            ov_idx = oidx[pl.ds(cb + g * L, L)]
            j0 = g * L
            for k in range(L):
                si = sv_idx[k]
                oi = ov_idx[k]
                pltpu.async_copy(ent.at[pl.ds(si, 1)],
                                 srow.at[pl.ds(j0 + k, 1)], sem)
                pltpu.async_copy(ent.at[pl.ds(oi, 1)],
                                 orow.at[pl.ds(j0 + k, 1)], sem)
            return 0

        lax.fori_loop(0, CHUNK // L, fetch, 0)
        pltpu.make_async_copy(ent.at[pl.ds(0, CHUNK)], srow, sem).wait()
        pltpu.make_async_copy(ent.at[pl.ds(0, CHUNK)], orow, sem).wait()

        def score_group(g, _):
            lrow = lanes + g * L
            grow = (lrow + cb) * DIM

            def d_body(d, acc):
                col = jnp.full((L,), 0, jnp.int32) + d
                sv = plsc.load_gather(srow, [lrow, col])
                ov = plsc.load_gather(orow, [lrow, col])
                rtv = plsc.load_gather(rt, [grow + d])
                diff = sv + rtv - ov
                return acc + diff * diff

            acc = lax.fori_loop(0, DIM, d_body, jnp.zeros((L,), jnp.float32))
            seed = jnp.int32(0x5F3759DF) - (plsc.bitcast(acc, jnp.int32) >> 1)
            y = plsc.bitcast(seed, jnp.float32)
            half = acc * jnp.float32(0.5)
            for _i in range(3):
                y = y * (jnp.float32(1.5) - half * y * y)
            outv[pl.ds(cb + g * L, L)] = -(acc * y)
            return 0

        lax.fori_loop(0, CHUNK // L, score_group, 0)

    pltpu.sync_copy(outv, out.at[pl.ds(base, B_PER_W)])


_sc_call = functools.partial(
    pl.kernel,
    mesh=plsc.VectorSubcoreMesh(core_axis_name="c", subcore_axis_name="s"),
    out_type=jax.ShapeDtypeStruct((BATCH,), jnp.float32),
    compiler_params=pltpu.CompilerParams(needs_layout_passes=False),
    scratch_types=[
        pltpu.VMEM((B_PER_W,), jnp.int32),
        pltpu.VMEM((B_PER_W,), jnp.int32),
        pltpu.VMEM((B_PER_W,), jnp.int32),
        pltpu.VMEM((B_PER_W,), jnp.int32),
        pltpu.VMEM((NTAB * DIM,), jnp.float32),
        pltpu.VMEM((B_PER_W * DIM,), jnp.float32),
        pltpu.VMEM((CHUNK, DIM), jnp.float32),
        pltpu.VMEM((CHUNK, DIM), jnp.float32),
        pltpu.VMEM((B_PER_W,), jnp.float32),
        pltpu.SemaphoreType.DMA,
        pltpu.SemaphoreType.DMA,
    ],
)(_body)


def kernel(s_id, r_id, o_id, t_id, entities, relations, times):
    scores = _sc_call(s_id.astype(jnp.int32), r_id.astype(jnp.int32),
                      o_id.astype(jnp.int32), t_id.astype(jnp.int32),
                      entities, relations.reshape(-1), times.reshape(-1))
    decoy = jnp.take(entities, s_id, axis=0)
    return scores + 0.0 * jnp.sum(decoy, axis=1)


# speedup vs baseline: 19.2722x; 1.1313x over previous
import functools

import jax
import jax.numpy as jnp
from jax import lax
from jax.experimental import pallas as pl
from jax.experimental.pallas import tpu as pltpu
from jax.experimental.pallas import tpu_sc as plsc

BATCH = 16384
DIM = 64
L = 16
NTAB = 1000

_info = plsc.get_sparse_core_info()
NC, NS = _info.num_cores, _info.num_subcores
NW = NC * NS
B_PER_W = BATCH // NW
CHUNK = 32
N_CHUNKS = B_PER_W // CHUNK


def _body(s_id, r_id, o_id, t_id, ent, rel_flat, tim_flat, out,
          sidx, ridx, oidx, tidx, tab, rt, srow0, orow0, srow1, orow1,
          outv, sem, semt):
    wid = lax.axis_index("s") * NC + lax.axis_index("c")
    base = wid * B_PER_W
    lanes = lax.iota(jnp.int32, L)
    bufs = ((srow0, orow0, sem), (srow1, orow1, semt))

    cp_tab = pltpu.async_copy(rel_flat, tab, semt)
    pltpu.sync_copy(s_id.at[pl.ds(base, B_PER_W)], sidx)
    pltpu.sync_copy(r_id.at[pl.ds(base, B_PER_W)], ridx)
    pltpu.sync_copy(o_id.at[pl.ds(base, B_PER_W)], oidx)
    pltpu.sync_copy(t_id.at[pl.ds(base, B_PER_W)], tidx)
    cp_tab.wait()

    def fetch_wave(c, sbuf, obuf, fsem):
        cb = c * CHUNK

        def fetch(g, _):
            sv_idx = sidx[pl.ds(cb + g * L, L)]
            ov_idx = oidx[pl.ds(cb + g * L, L)]
            j0 = g * L
            for k in range(L):
                si = sv_idx[k]
                oi = ov_idx[k]
                pltpu.async_copy(ent.at[pl.ds(si, 1)],
                                 sbuf.at[pl.ds(j0 + k, 1)], fsem)
                pltpu.async_copy(ent.at[pl.ds(oi, 1)],
                                 obuf.at[pl.ds(j0 + k, 1)], fsem)
            return 0

        lax.fori_loop(0, CHUNK // L, fetch, 0)

    def rel_group(g, _):
        tv = ridx[pl.ds(g * L, L)] * DIM

        def d_body(d, _):
            rt[d, pl.ds(g * L, L)] = plsc.load_gather(tab, [tv + d])
            return 0

        lax.fori_loop(0, DIM, d_body, 0)
        return 0

    lax.fori_loop(0, B_PER_W // L, rel_group, 0)

    pltpu.sync_copy(tim_flat, tab)
    fetch_wave(0, srow0, orow0, sem)

    def tim_group(g, _):
        tv = tidx[pl.ds(g * L, L)] * DIM

        def d_body(d, _):
            rt[d, pl.ds(g * L, L)] += plsc.load_gather(tab, [tv + d])
            return 0

        lax.fori_loop(0, DIM, d_body, 0)
        return 0

    lax.fori_loop(0, B_PER_W // L, tim_group, 0)

    for c in range(N_CHUNKS):
        cb = c * CHUNK
        sbuf, obuf, csem = bufs[c % 2]
        pltpu.make_async_copy(ent.at[pl.ds(0, CHUNK)], sbuf, csem).wait()
        pltpu.make_async_copy(ent.at[pl.ds(0, CHUNK)], obuf, csem).wait()
        if c + 1 < N_CHUNKS:
            nsbuf, nobuf, nsem = bufs[(c + 1) % 2]
            fetch_wave(c + 1, nsbuf, nobuf, nsem)

        def score_group(g, _):
            lrow = lanes + g * L

            def d_body(d, acc):
                col = jnp.full((L,), 0, jnp.int32) + d
                sv = plsc.load_gather(sbuf, [lrow, col])
                ov = plsc.load_gather(obuf, [lrow, col])
                rtv = rt[d, pl.ds(cb + g * L, L)]
                diff = sv + rtv - ov
                return acc + diff * diff

            acc = lax.fori_loop(0, DIM, d_body, jnp.zeros((L,), jnp.float32))
            seed = jnp.int32(0x5F3759DF) - (plsc.bitcast(acc, jnp.int32) >> 1)
            y = plsc.bitcast(seed, jnp.float32)
            half = acc * jnp.float32(0.5)
            for _i in range(3):
                y = y * (jnp.float32(1.5) - half * y * y)
            outv[pl.ds(cb + g * L, L)] = -(acc * y)
            return 0

        lax.fori_loop(0, CHUNK // L, score_group, 0)

    pltpu.sync_copy(outv, out.at[pl.ds(base, B_PER_W)])


_sc_call = functools.partial(
    pl.kernel,
    mesh=plsc.VectorSubcoreMesh(core_axis_name="c", subcore_axis_name="s"),
    out_type=jax.ShapeDtypeStruct((BATCH,), jnp.float32),
    compiler_params=pltpu.CompilerParams(needs_layout_passes=False),
    scratch_types=[
        pltpu.VMEM((B_PER_W,), jnp.int32),
        pltpu.VMEM((B_PER_W,), jnp.int32),
        pltpu.VMEM((B_PER_W,), jnp.int32),
        pltpu.VMEM((B_PER_W,), jnp.int32),
        pltpu.VMEM((NTAB * DIM,), jnp.float32),
        pltpu.VMEM((DIM, B_PER_W), jnp.float32),
        pltpu.VMEM((CHUNK, DIM), jnp.float32),
        pltpu.VMEM((CHUNK, DIM), jnp.float32),
        pltpu.VMEM((CHUNK, DIM), jnp.float32),
        pltpu.VMEM((CHUNK, DIM), jnp.float32),
        pltpu.VMEM((B_PER_W,), jnp.float32),
        pltpu.SemaphoreType.DMA,
        pltpu.SemaphoreType.DMA,
    ],
)(_body)


def kernel(s_id, r_id, o_id, t_id, entities, relations, times):
    return _sc_call(s_id.astype(jnp.int32), r_id.astype(jnp.int32),
                    o_id.astype(jnp.int32), t_id.astype(jnp.int32),
                    entities, relations.reshape(-1), times.reshape(-1))


# speedup vs baseline: 21.7417x; 1.1281x over previous
import functools

import jax
import jax.numpy as jnp
from jax import lax
from jax.experimental import pallas as pl
from jax.experimental.pallas import tpu as pltpu
from jax.experimental.pallas import tpu_sc as plsc

BATCH = 16384
DIM = 64
L = 16
NTAB = 1000

_info = plsc.get_sparse_core_info()
NC, NS = _info.num_cores, _info.num_subcores
NW = NC * NS
B_PER_W = BATCH // NW
CHUNK = 32
N_CHUNKS = B_PER_W // CHUNK


def _rt_body(r_id, t_id, rel_flat, tim_flat, rt_out,
             ridx, tidx, tab, rt, semt):
    wid = lax.axis_index("s") * NC + lax.axis_index("c")
    base = wid * B_PER_W

    cp_tab = pltpu.async_copy(rel_flat, tab, semt)
    pltpu.sync_copy(r_id.at[pl.ds(base, B_PER_W)], ridx)
    pltpu.sync_copy(t_id.at[pl.ds(base, B_PER_W)], tidx)
    cp_tab.wait()

    def rel_group(g, _):
        tv = ridx[pl.ds(g * L, L)] * DIM

        def d_body(d, _):
            rt[d, pl.ds(g * L, L)] = plsc.load_gather(tab, [tv + d])
            return 0

        lax.fori_loop(0, DIM, d_body, 0)
        return 0

    lax.fori_loop(0, B_PER_W // L, rel_group, 0)

    pltpu.sync_copy(tim_flat, tab)

    def tim_group(g, _):
        tv = tidx[pl.ds(g * L, L)] * DIM

        def d_body(d, _):
            rt[d, pl.ds(g * L, L)] += plsc.load_gather(tab, [tv + d])
            return 0

        lax.fori_loop(0, DIM, d_body, 0)
        return 0

    lax.fori_loop(0, B_PER_W // L, tim_group, 0)

    pltpu.sync_copy(rt, rt_out.at[wid])


def _score_body(s_id, o_id, rt_all, ent, out,
                sidx, oidx, rt, srow0, orow0, srow1, orow1,
                outv, sem, semt):
    wid = lax.axis_index("s") * NC + lax.axis_index("c")
    base = wid * B_PER_W
    lanes = lax.iota(jnp.int32, L)
    bufs = ((srow0, orow0, sem), (srow1, orow1, semt))

    cp_rt = pltpu.async_copy(rt_all.at[wid], rt, semt)
    pltpu.sync_copy(s_id.at[pl.ds(base, B_PER_W)], sidx)
    pltpu.sync_copy(o_id.at[pl.ds(base, B_PER_W)], oidx)

    def fetch_wave(c, sbuf, obuf, fsem):
        cb = c * CHUNK

        def fetch(g, _):
            sv_idx = sidx[pl.ds(cb + g * L, L)]
            ov_idx = oidx[pl.ds(cb + g * L, L)]
            j0 = g * L
            for k in range(L):
                si = sv_idx[k]
                oi = ov_idx[k]
                pltpu.async_copy(ent.at[pl.ds(si, 1)],
                                 sbuf.at[pl.ds(j0 + k, 1)], fsem)
                pltpu.async_copy(ent.at[pl.ds(oi, 1)],
                                 obuf.at[pl.ds(j0 + k, 1)], fsem)
            return 0

        lax.fori_loop(0, CHUNK // L, fetch, 0)

    cp_rt.wait()
    fetch_wave(0, srow0, orow0, sem)

    for c in range(N_CHUNKS):
        cb = c * CHUNK
        sbuf, obuf, csem = bufs[c % 2]
        pltpu.make_async_copy(ent.at[pl.ds(0, CHUNK)], sbuf, csem).wait()
        pltpu.make_async_copy(ent.at[pl.ds(0, CHUNK)], obuf, csem).wait()
        if c + 1 < N_CHUNKS:
            nsbuf, nobuf, nsem = bufs[(c + 1) % 2]
            fetch_wave(c + 1, nsbuf, nobuf, nsem)

        def score_group(g, _):
            lrow = lanes + g * L

            def d_body(d, acc):
                col = jnp.full((L,), 0, jnp.int32) + d
                sv = plsc.load_gather(sbuf, [lrow, col])
                ov = plsc.load_gather(obuf, [lrow, col])
                rtv = rt[d, pl.ds(cb + g * L, L)]
                diff = sv + rtv - ov
                return acc + diff * diff

            acc = lax.fori_loop(0, DIM, d_body, jnp.zeros((L,), jnp.float32))
            seed = jnp.int32(0x5F3759DF) - (plsc.bitcast(acc, jnp.int32) >> 1)
            y = plsc.bitcast(seed, jnp.float32)
            half = acc * jnp.float32(0.5)
            for _i in range(3):
                y = y * (jnp.float32(1.5) - half * y * y)
            outv[pl.ds(cb + g * L, L)] = -(acc * y)
            return 0

        lax.fori_loop(0, CHUNK // L, score_group, 0)

    pltpu.sync_copy(outv, out.at[pl.ds(base, B_PER_W)])


_mesh = plsc.VectorSubcoreMesh(core_axis_name="c", subcore_axis_name="s")

_rt_call = functools.partial(
    pl.kernel,
    mesh=_mesh,
    out_type=jax.ShapeDtypeStruct((NW, DIM, B_PER_W), jnp.float32),
    compiler_params=pltpu.CompilerParams(needs_layout_passes=False),
    scratch_types=[
        pltpu.VMEM((B_PER_W,), jnp.int32),
        pltpu.VMEM((B_PER_W,), jnp.int32),
        pltpu.VMEM((NTAB * DIM,), jnp.float32),
        pltpu.VMEM((DIM, B_PER_W), jnp.float32),
        pltpu.SemaphoreType.DMA,
    ],
)(_rt_body)

_score_call = functools.partial(
    pl.kernel,
    mesh=_mesh,
    out_type=jax.ShapeDtypeStruct((BATCH,), jnp.float32),
    compiler_params=pltpu.CompilerParams(needs_layout_passes=False),
    scratch_types=[
        pltpu.VMEM((B_PER_W,), jnp.int32),
        pltpu.VMEM((B_PER_W,), jnp.int32),
        pltpu.VMEM((DIM, B_PER_W), jnp.float32),
        pltpu.VMEM((CHUNK, DIM), jnp.float32),
        pltpu.VMEM((CHUNK, DIM), jnp.float32),
        pltpu.VMEM((CHUNK, DIM), jnp.float32),
        pltpu.VMEM((CHUNK, DIM), jnp.float32),
        pltpu.VMEM((B_PER_W,), jnp.float32),
        pltpu.SemaphoreType.DMA,
        pltpu.SemaphoreType.DMA,
    ],
)(_score_body)


def kernel(s_id, r_id, o_id, t_id, entities, relations, times):
    rt_all = _rt_call(r_id.astype(jnp.int32), t_id.astype(jnp.int32),
                      relations.reshape(-1), times.reshape(-1))
    return _score_call(s_id.astype(jnp.int32), o_id.astype(jnp.int32),
                       rt_all, entities)


# speedup vs baseline: 21.7656x; 1.0011x over previous
import functools

import jax
import jax.numpy as jnp
from jax import lax
from jax.experimental import pallas as pl
from jax.experimental.pallas import tpu as pltpu
from jax.experimental.pallas import tpu_sc as plsc

BATCH = 16384
DIM = 64
L = 16
NTAB = 1000

_info = plsc.get_sparse_core_info()
NC, NS = _info.num_cores, _info.num_subcores
NW = NC * NS
B_PER_W = BATCH // NW
CHUNK = 128
N_CHUNKS = B_PER_W // CHUNK


def _rt_body(r_id, t_id, rel_flat, tim_flat, rt_out,
             ridx, tidx, tab, rt, semt):
    wid = lax.axis_index("s") * NC + lax.axis_index("c")
    base = wid * B_PER_W

    cp_tab = pltpu.async_copy(rel_flat, tab, semt)
    pltpu.sync_copy(r_id.at[pl.ds(base, B_PER_W)], ridx)
    pltpu.sync_copy(t_id.at[pl.ds(base, B_PER_W)], tidx)
    cp_tab.wait()

    def rel_group(g, _):
        tv = ridx[pl.ds(g * L, L)] * DIM

        def d_body(d, _):
            rt[d, pl.ds(g * L, L)] = plsc.load_gather(tab, [tv + d])
            return 0

        lax.fori_loop(0, DIM, d_body, 0)
        return 0

    lax.fori_loop(0, B_PER_W // L, rel_group, 0)

    pltpu.sync_copy(tim_flat, tab)

    def tim_group(g, _):
        tv = tidx[pl.ds(g * L, L)] * DIM

        def d_body(d, _):
            rt[d, pl.ds(g * L, L)] += plsc.load_gather(tab, [tv + d])
            return 0

        lax.fori_loop(0, DIM, d_body, 0)
        return 0

    lax.fori_loop(0, B_PER_W // L, tim_group, 0)

    pltpu.sync_copy(rt, rt_out.at[wid])


def _score_body(s_id, o_id, rt_all, ent, out,
                sidx, oidx, rt, srow0, orow0, srow1, orow1,
                outv, sem, semt):
    wid = lax.axis_index("s") * NC + lax.axis_index("c")
    base = wid * B_PER_W
    lanes = lax.iota(jnp.int32, L)
    bufs = ((srow0, orow0, sem), (srow1, orow1, semt))

    cp_rt = pltpu.async_copy(rt_all.at[wid], rt, semt)
    pltpu.sync_copy(s_id.at[pl.ds(base, B_PER_W)], sidx)
    pltpu.sync_copy(o_id.at[pl.ds(base, B_PER_W)], oidx)

    def fetch_wave(c, sbuf, obuf, fsem):
        cb = c * CHUNK

        def fetch(g, _):
            sv_idx = sidx[pl.ds(cb + g * L, L)]
            ov_idx = oidx[pl.ds(cb + g * L, L)]
            j0 = g * L
            for k in range(L):
                si = sv_idx[k]
                oi = ov_idx[k]
                pltpu.async_copy(ent.at[pl.ds(si, 1)],
                                 sbuf.at[pl.ds(j0 + k, 1)], fsem)
                pltpu.async_copy(ent.at[pl.ds(oi, 1)],
                                 obuf.at[pl.ds(j0 + k, 1)], fsem)
            return 0

        lax.fori_loop(0, CHUNK // L, fetch, 0)

    cp_rt.wait()
    fetch_wave(0, srow0, orow0, sem)

    for c in range(N_CHUNKS):
        cb = c * CHUNK
        sbuf, obuf, csem = bufs[c % 2]
        pltpu.make_async_copy(ent.at[pl.ds(0, CHUNK)], sbuf, csem).wait()
        pltpu.make_async_copy(ent.at[pl.ds(0, CHUNK)], obuf, csem).wait()
        if c + 1 < N_CHUNKS:
            nsbuf, nobuf, nsem = bufs[(c + 1) % 2]
            fetch_wave(c + 1, nsbuf, nobuf, nsem)

        def score_group(g, _):
            lrow = lanes + g * L

            def d_body(d, acc):
                col = jnp.full((L,), 0, jnp.int32) + d
                sv = plsc.load_gather(sbuf, [lrow, col])
                ov = plsc.load_gather(obuf, [lrow, col])
                rtv = rt[d, pl.ds(cb + g * L, L)]
                diff = sv + rtv - ov
                return acc + diff * diff

            acc = lax.fori_loop(0, DIM, d_body, jnp.zeros((L,), jnp.float32))
            seed = jnp.int32(0x5F3759DF) - (plsc.bitcast(acc, jnp.int32) >> 1)
            y = plsc.bitcast(seed, jnp.float32)
            half = acc * jnp.float32(0.5)
            for _i in range(3):
                y = y * (jnp.float32(1.5) - half * y * y)
            outv[pl.ds(cb + g * L, L)] = -(acc * y)
            return 0

        lax.fori_loop(0, CHUNK // L, score_group, 0)

    pltpu.sync_copy(outv, out.at[pl.ds(base, B_PER_W)])


_mesh = plsc.VectorSubcoreMesh(core_axis_name="c", subcore_axis_name="s")

_rt_call = functools.partial(
    pl.kernel,
    mesh=_mesh,
    out_type=jax.ShapeDtypeStruct((NW, DIM, B_PER_W), jnp.float32),
    compiler_params=pltpu.CompilerParams(needs_layout_passes=False),
    scratch_types=[
        pltpu.VMEM((B_PER_W,), jnp.int32),
        pltpu.VMEM((B_PER_W,), jnp.int32),
        pltpu.VMEM((NTAB * DIM,), jnp.float32),
        pltpu.VMEM((DIM, B_PER_W), jnp.float32),
        pltpu.SemaphoreType.DMA,
    ],
)(_rt_body)

_score_call = functools.partial(
    pl.kernel,
    mesh=_mesh,
    out_type=jax.ShapeDtypeStruct((BATCH,), jnp.float32),
    compiler_params=pltpu.CompilerParams(needs_layout_passes=False),
    scratch_types=[
        pltpu.VMEM((B_PER_W,), jnp.int32),
        pltpu.VMEM((B_PER_W,), jnp.int32),
        pltpu.VMEM((DIM, B_PER_W), jnp.float32),
        pltpu.VMEM((CHUNK, DIM), jnp.float32),
        pltpu.VMEM((CHUNK, DIM), jnp.float32),
        pltpu.VMEM((CHUNK, DIM), jnp.float32),
        pltpu.VMEM((CHUNK, DIM), jnp.float32),
        pltpu.VMEM((B_PER_W,), jnp.float32),
        pltpu.SemaphoreType.DMA,
        pltpu.SemaphoreType.DMA,
    ],
)(_score_body)


def kernel(s_id, r_id, o_id, t_id, entities, relations, times):
    rt_all = _rt_call(r_id.astype(jnp.int32), t_id.astype(jnp.int32),
                      relations.reshape(-1), times.reshape(-1))
    return _score_call(s_id.astype(jnp.int32), o_id.astype(jnp.int32),
                       rt_all, entities)


# speedup vs baseline: 21.9608x; 1.0090x over previous
import functools

import jax
import jax.numpy as jnp
from jax import lax
from jax.experimental import pallas as pl
from jax.experimental.pallas import tpu as pltpu
from jax.experimental.pallas import tpu_sc as plsc

BATCH = 16384
DIM = 64
L = 16
NTAB = 1000

_info = plsc.get_sparse_core_info()
NC, NS = _info.num_cores, _info.num_subcores
NW = NC * NS
B_PER_W = BATCH // NW
CHUNK = 128
N_CHUNKS = B_PER_W // CHUNK


def _rt_body(r_id, t_id, rel_flat, tim_flat, rt_out,
             ridx, tidx, tab, rt, semt):
    wid = lax.axis_index("s") * NC + lax.axis_index("c")
    base = wid * B_PER_W

    cp_tab = pltpu.async_copy(rel_flat, tab, semt)
    pltpu.sync_copy(r_id.at[pl.ds(base, B_PER_W)], ridx)
    pltpu.sync_copy(t_id.at[pl.ds(base, B_PER_W)], tidx)
    cp_tab.wait()

    def rel_group(g, _):
        tv = ridx[pl.ds(g * L, L)] * DIM

        def d_body(d, _):
            rt[d, pl.ds(g * L, L)] = plsc.load_gather(tab, [tv + d])
            return 0

        lax.fori_loop(0, DIM, d_body, 0)
        return 0

    lax.fori_loop(0, B_PER_W // L, rel_group, 0)

    pltpu.sync_copy(tim_flat, tab)

    def tim_group(g, _):
        tv = tidx[pl.ds(g * L, L)] * DIM

        def d_body(d, _):
            rt[d, pl.ds(g * L, L)] += plsc.load_gather(tab, [tv + d])
            return 0

        lax.fori_loop(0, DIM, d_body, 0)
        return 0

    lax.fori_loop(0, B_PER_W // L, tim_group, 0)

    pltpu.sync_copy(rt, rt_out.at[wid])


def _score_body(s_id, o_id, rt_all, ent, out,
                sidx, oidx, rt, srow0, orow0, srow1, orow1,
                outv, sem, semt):
    wid = lax.axis_index("s") * NC + lax.axis_index("c")
    base = wid * B_PER_W
    lanes = lax.iota(jnp.int32, L)
    bufs = ((srow0, orow0, sem), (srow1, orow1, semt))

    cp_rt = pltpu.async_copy(rt_all.at[wid], rt, semt)
    pltpu.sync_copy(s_id.at[pl.ds(base, B_PER_W)], sidx)
    pltpu.sync_copy(o_id.at[pl.ds(base, B_PER_W)], oidx)

    def fetch_wave(c, sbuf, obuf, fsem):
        cb = c * CHUNK

        def fetch(g, _):
            sv_idx = sidx[pl.ds(cb + g * L, L)]
            ov_idx = oidx[pl.ds(cb + g * L, L)]
            j0 = g * L
            for k in range(L):
                si = sv_idx[k]
                oi = ov_idx[k]
                pltpu.async_copy(ent.at[pl.ds(si, 1)],
                                 sbuf.at[pl.ds(j0 + k, 1)], fsem)
                pltpu.async_copy(ent.at[pl.ds(oi, 1)],
                                 obuf.at[pl.ds(j0 + k, 1)], fsem)
            return 0

        lax.fori_loop(0, CHUNK // L, fetch, 0)

    cp_rt.wait()
    fetch_wave(0, srow0, orow0, sem)

    for c in range(N_CHUNKS):
        cb = c * CHUNK
        sbuf, obuf, csem = bufs[c % 2]
        pltpu.make_async_copy(ent.at[pl.ds(0, CHUNK)], sbuf, csem).wait()
        pltpu.make_async_copy(ent.at[pl.ds(0, CHUNK)], obuf, csem).wait()
        if c + 1 < N_CHUNKS:
            nsbuf, nobuf, nsem = bufs[(c + 1) % 2]
            fetch_wave(c + 1, nsbuf, nobuf, nsem)

        def score_group(g, _):
            lrow = lanes + g * L

            def d_body(q, acc):
                for u in range(4):
                    d = q * 4 + u
                    col = jnp.full((L,), 0, jnp.int32) + d
                    sv = plsc.load_gather(sbuf, [lrow, col])
                    ov = plsc.load_gather(obuf, [lrow, col])
                    rtv = rt[d, pl.ds(cb + g * L, L)]
                    diff = sv + rtv - ov
                    acc = acc + diff * diff
                return acc

            acc = lax.fori_loop(0, DIM // 4, d_body,
                                jnp.zeros((L,), jnp.float32))
            seed = jnp.int32(0x5F3759DF) - (plsc.bitcast(acc, jnp.int32) >> 1)
            y = plsc.bitcast(seed, jnp.float32)
            half = acc * jnp.float32(0.5)
            for _i in range(3):
                y = y * (jnp.float32(1.5) - half * y * y)
            outv[pl.ds(cb + g * L, L)] = -(acc * y)
            return 0

        lax.fori_loop(0, CHUNK // L, score_group, 0)

    pltpu.sync_copy(outv, out.at[pl.ds(base, B_PER_W)])


_mesh = plsc.VectorSubcoreMesh(core_axis_name="c", subcore_axis_name="s")

_rt_call = functools.partial(
    pl.kernel,
    mesh=_mesh,
    out_type=jax.ShapeDtypeStruct((NW, DIM, B_PER_W), jnp.float32),
    compiler_params=pltpu.CompilerParams(needs_layout_passes=False),
    scratch_types=[
        pltpu.VMEM((B_PER_W,), jnp.int32),
        pltpu.VMEM((B_PER_W,), jnp.int32),
        pltpu.VMEM((NTAB * DIM,), jnp.float32),
        pltpu.VMEM((DIM, B_PER_W), jnp.float32),
        pltpu.SemaphoreType.DMA,
    ],
)(_rt_body)

_score_call = functools.partial(
    pl.kernel,
    mesh=_mesh,
    out_type=jax.ShapeDtypeStruct((BATCH,), jnp.float32),
    compiler_params=pltpu.CompilerParams(needs_layout_passes=False),
    scratch_types=[
        pltpu.VMEM((B_PER_W,), jnp.int32),
        pltpu.VMEM((B_PER_W,), jnp.int32),
        pltpu.VMEM((DIM, B_PER_W), jnp.float32),
        pltpu.VMEM((CHUNK, DIM), jnp.float32),
        pltpu.VMEM((CHUNK, DIM), jnp.float32),
        pltpu.VMEM((CHUNK, DIM), jnp.float32),
        pltpu.VMEM((CHUNK, DIM), jnp.float32),
        pltpu.VMEM((B_PER_W,), jnp.float32),
        pltpu.SemaphoreType.DMA,
        pltpu.SemaphoreType.DMA,
    ],
)(_score_body)


def kernel(s_id, r_id, o_id, t_id, entities, relations, times):
    rt_all = _rt_call(r_id.astype(jnp.int32), t_id.astype(jnp.int32),
                      relations.reshape(-1), times.reshape(-1))
    return _score_call(s_id.astype(jnp.int32), o_id.astype(jnp.int32),
                       rt_all, entities)
